# Initial kernel scaffold; baseline (speedup 1.0000x reference)
import jax
import jax.numpy as jnp
from jax.experimental import pallas as pl


def kernel(goal_feature, cap_feature, img_feature, Wg_w, Wg_b, end_w, end_b, ia_Wq, ia_bq, ia_Wk, ia_bk, ia_v, ca_Wq, ca_bq, ca_Wk, ca_bk, ca_v, c0_Wl, c0_Wr, c0_att, c0_b, c1_Wl, c1_Wr, c1_att, c1_b, cap_emb_mask, edge_index):
    raise NotImplementedError("write your pallas kernel here")


# speedup vs baseline: 1.0003x; 1.0003x over previous
import functools
import jax
import jax.numpy as jnp
from jax import lax
from jax.experimental import pallas as pl
from jax.experimental.pallas import tpu as pltpu

D = 256
H = 4
Ch = 64
N = 1024
BLK = 128


def _att_pool_body(feat_ref, q_ref, Wq_ref, bq_ref, v_ref, mask_ref,
                   w_ref, aggr_ref):
    feat = feat_ref[...]
    q = q_ref[...]
    Wq = Wq_ref[...]
    bq = bq_ref[...]
    v = v_ref[...]
    mask = mask_ref[...]
    h = lax.dot_general(feat, Wq, (((2,), (0,)), ((), ())),
                        preferred_element_type=jnp.float32)
    h = jnp.tanh(h + bq[0][None, None, :] + q[0][None, None, :])
    scores = jnp.sum(h * v[0][None, None, :], axis=2)
    scores = jnp.where(mask > 0.5, scores, -1e9)
    m = jnp.max(scores, axis=1, keepdims=True)
    ex = jnp.exp(scores - m)
    w = ex / jnp.sum(ex, axis=1, keepdims=True)
    w_ref[...] = w
    aggr_ref[...] = jnp.sum(feat * w[:, :, None], axis=1)


def _att_pool(feat_p, q, Wq, bq, v, mask_p, T_pad):
    Bp = feat_p.shape[0]
    grid = (Bp // BLK,)
    return pl.pallas_call(
        _att_pool_body,
        grid=grid,
        in_specs=[
            pl.BlockSpec((BLK, T_pad, D), lambda i: (i, 0, 0)),
            pl.BlockSpec((1, 64), lambda i: (0, 0)),
            pl.BlockSpec((D, 64), lambda i: (0, 0)),
            pl.BlockSpec((1, 64), lambda i: (0, 0)),
            pl.BlockSpec((1, 64), lambda i: (0, 0)),
            pl.BlockSpec((BLK, T_pad), lambda i: (i, 0)),
        ],
        out_specs=[
            pl.BlockSpec((BLK, T_pad), lambda i: (i, 0)),
            pl.BlockSpec((BLK, D), lambda i: (i, 0)),
        ],
        out_shape=[
            jax.ShapeDtypeStruct((Bp, T_pad), jnp.float32),
            jax.ShapeDtypeStruct((Bp, D), jnp.float32),
        ],
    )(feat_p, q, Wq, bq, v, mask_p)


def _gatv2_edge(x, src, dst, Wl, Wr, att, b):
    xl = (x @ Wl).reshape(N, H, Ch)
    xr = (x @ Wr).reshape(N, H, Ch)
    e = jax.nn.leaky_relu(xl[src] + xr[dst], negative_slope=0.2)
    logit = jnp.sum(e * att[None, :, :], axis=-1)
    m = jax.ops.segment_max(logit, dst, num_segments=N)
    m = jnp.where(jnp.isfinite(m), m, 0.0)
    ex = jnp.exp(logit - m[dst])
    s = jax.ops.segment_sum(ex, dst, num_segments=N)
    alpha = ex / (s[dst] + 1e-16)
    out = jax.ops.segment_sum(xl[src] * alpha[:, :, None], dst, num_segments=N)
    return out.reshape(N, H * Ch) + b, alpha


def _dense_adj(src, dst, alpha):
    adj = jnp.zeros((N, N, H), jnp.float32)
    return adj.at[src, dst].add(alpha)


def kernel(goal_feature, cap_feature, img_feature, Wg_w, Wg_b, end_w, end_b,
           ia_Wq, ia_bq, ia_Wk, ia_bk, ia_v, ca_Wq, ca_bq, ca_Wk, ca_bk, ca_v,
           c0_Wl, c0_Wr, c0_att, c0_b, c1_Wl, c1_Wr, c1_att, c1_b,
           cap_emb_mask, edge_index):
    convs = [(c0_Wl, c0_Wr, c0_att, c0_b), (c1_Wl, c1_Wr, c1_att, c1_b)]
    src = edge_index[0]
    dst = edge_index[1]
    B = cap_feature.shape[0]
    Bp = N

    def prep(feat, mask):
        T = feat.shape[1]
        T_pad = ((T + 63) // 64) * 64
        feat_p = jnp.zeros((Bp, T_pad, D), jnp.float32).at[:B, :T].set(feat)
        m = jnp.zeros((Bp, T_pad), jnp.float32)
        if mask is None:
            m = m.at[:B, :T].set(1.0)
        else:
            m = m.at[:B, :T].set(mask.astype(jnp.float32))
        return feat_p, m, T, T_pad

    def branch(feat, Wq, bq, Wk, bk, v, mask):
        feat_p, mask_p, T, T_pad = prep(feat, mask)
        bq2 = bq[None, :]
        v2 = v[None, :]
        w_list = []
        adj_list = []
        node = None
        for idx in range(2):
            if idx == 0:
                cg = goal_feature @ Wg_w + Wg_b
                ce = goal_feature @ end_w + end_b
            else:
                cg = node[0:1]
                ce = node[1:2]
            q = cg @ Wk + bk[None, :]
            w_p, aggr_p = _att_pool(feat_p, q, Wq, bq2, v2, mask_p, T_pad)
            w = w_p[:B, :T]
            aggr = aggr_p[:B]
            prev = jnp.concatenate([cg, ce, aggr], axis=0)
            out, alpha = _gatv2_edge(prev, src, dst, *convs[idx])
            node = jax.nn.elu(out + prev)
            w_list.append(w)
            adj_list.append(_dense_adj(src, dst, alpha))
        return jnp.stack(w_list), jnp.stack(adj_list)

    img_w, img_adj = branch(img_feature, ia_Wq, ia_bq, ia_Wk, ia_bk, ia_v, None)
    cap_w, cap_adj = branch(cap_feature, ca_Wq, ca_bq, ca_Wk, ca_bk, ca_v,
                            cap_emb_mask)
    return (img_w, img_adj, cap_w, cap_adj)
